# baseline (speedup 1.0000x reference)
import functools

import jax
import jax.numpy as jnp
from jax import lax
from jax.experimental import pallas as pl
from jax.experimental.pallas import tpu as pltpu
from jax.experimental.pallas import tpu_sc as plsc

LANES = 128
NW = 32


NBUF = 2
IBUF = 4

FASTC = 1
GPF = 120
GPS = 40


def _sc_aggregate(node, e2, n_acc, gpf, gps):
    n, d = node.shape
    rpt = n_acc // 16
    nzc = rpt // LANES
    mesh = plsc.VectorSubcoreMesh(core_axis_name="c", subcore_axis_name="s")

    @functools.partial(
        pl.kernel,
        out_type=jax.ShapeDtypeStruct((2, n_acc, d), jnp.float32),
        mesh=mesh,
        scratch_types=[
            pltpu.VMEM((IBUF, 2, LANES), jnp.int32),
            pltpu.VMEM((NBUF, LANES, d), jnp.float32),
            pltpu.VMEM_SHARED((n_acc, d), jnp.float32),
        ] + [pltpu.SemaphoreType.DMA] * (NBUF + IBUF),
    )
    def k(node_hbm, e2_hbm, out_hbm, idx, rows, aggr, *sems):
        gsem = sems[:NBUF]
        isem = sems[NBUF:]
        c = lax.axis_index("c")
        s = lax.axis_index("s")
        fast = c == FASTC
        gcnt = jnp.where(fast, gpf, gps)
        gbase = jnp.where(fast, s * gpf, 16 * gpf + s * gps)

        for q in range(IBUF):
            pltpu.async_copy(e2_hbm.at[gbase + q], idx.at[q], isem[q])

        def zrow(r, carry):
            for j in range(d // 16):
                rows[0, r, pl.ds(j * 16, 16)] = jnp.zeros((16,), jnp.float32)
            return carry
        lax.fori_loop(0, LANES, zrow, 0)
        for kk in range(nzc):
            pltpu.sync_copy(rows.at[0],
                            aggr.at[pl.ds(s * rpt + kk * LANES, LANES)])

        for b in range(NBUF):
            pltpu.make_async_copy(e2_hbm.at[gbase + b], idx.at[b],
                                  isem[b]).wait()
            pltpu.async_copy(node_hbm.at[idx.at[b, 0]], rows.at[b], gsem[b])
        plsc.subcore_barrier()

        def step(g, j, fetch_ahead, gather_ahead):
            b = j % NBUF
            q = j % IBUF
            pltpu.make_async_copy(node_hbm.at[idx.at[q, 0]],
                                  rows.at[b], gsem[b]).wait()
            pltpu.sync_copy(rows.at[b], aggr.at[idx.at[q, 1]], add=True)
            if fetch_ahead:
                pltpu.async_copy(e2_hbm.at[gbase + g + IBUF], idx.at[q],
                                 isem[q])
            if gather_ahead:
                q2 = (j + NBUF) % IBUF
                pltpu.make_async_copy(e2_hbm.at[gbase + g + NBUF],
                                      idx.at[q2], isem[q2]).wait()
                pltpu.async_copy(node_hbm.at[idx.at[q2, 0]], rows.at[b],
                                 gsem[b])

        def body(kk, carry):
            for j in range(IBUF):
                step(kk * IBUF + j, j, True, True)
            return carry
        lax.fori_loop(0, gcnt // IBUF - 1, body, 0)
        for j in range(IBUF):
            g = gcnt - IBUF + j
            step(g, j, False, j + NBUF < IBUF)

        plsc.subcore_barrier()
        pltpu.sync_copy(aggr.at[pl.ds(s * rpt, rpt)],
                        out_hbm.at[c, pl.ds(s * rpt, rpt)])

    return k(node, e2)


def _mlp_block(node_ref, p0_ref, p1_ref, w1_ref, w2_ref, w3_ref, v_ref,
               eps_ref, out_ref):
    def ln(x, g, b):
        mu = jnp.mean(x, axis=-1, keepdims=True)
        var = jnp.mean((x - mu) ** 2, axis=-1, keepdims=True)
        return (x - mu) * lax.rsqrt(var + 1e-5) * g + b

    b1 = v_ref[0:1, :]
    g1 = v_ref[1:2, :]
    be1 = v_ref[2:3, :]
    b2 = v_ref[3:4, :]
    g2 = v_ref[4:5, :]
    be2 = v_ref[5:6, :]
    b3 = v_ref[6:7, :]
    gn = v_ref[7:8, :]
    bn = v_ref[8:9, :]

    h = (1.0 + eps_ref[0]) * node_ref[...] + p0_ref[0] + p1_ref[0]
    x = jnp.dot(h, w1_ref[...], preferred_element_type=jnp.float32) + b1
    x = jnp.maximum(ln(x, g1, be1), 0.0)
    x = jnp.dot(x, w2_ref[...], preferred_element_type=jnp.float32) + b2
    x = jnp.maximum(ln(x, g2, be2), 0.0)
    x = jnp.dot(x, w3_ref[...], preferred_element_type=jnp.float32) + b3
    out_ref[...] = jnp.maximum(ln(x, gn, bn), 0.0)


def kernel(node, edge_index, edge_attr, batch_ptr, W1, b1, g1, be1,
           W2, b2, g2, be2, W3, b3, eps, gN, bN):
    n, d = node.shape
    e = edge_index.shape[1]

    scale = -(-e // (16 * (GPF + GPS) * LANES))
    gpf, gps = GPF * scale, GPS * scale
    gt = 16 * (gpf + gps)
    pad = gt * LANES - e
    src = edge_index[0].astype(jnp.int32)
    dst = edge_index[1].astype(jnp.int32)
    if pad:
        src = jnp.concatenate([src, jnp.zeros((pad,), jnp.int32)])
        dst = jnp.concatenate([dst, jnp.full((pad,), n, jnp.int32)])
    e2 = jnp.stack([src.reshape(gt, LANES), dst.reshape(gt, LANES)], axis=1)

    n_acc = -(-(n + 1) // (16 * LANES)) * (16 * LANES)

    partials = _sc_aggregate(node, e2, n_acc, gpf, gps)

    rb = 1000
    grid = (n // rb,)
    vecs = jnp.stack([b1, g1, be1, b2, g2, be2, b3, gN, bN])
    eps2 = jnp.reshape(eps, (1,))

    out = pl.pallas_call(
        _mlp_block,
        grid=grid,
        in_specs=[
            pl.BlockSpec((rb, d), lambda i: (i, 0)),
            pl.BlockSpec((1, rb, d), lambda i: (0, i, 0)),
            pl.BlockSpec((1, rb, d), lambda i: (1, i, 0)),
            pl.BlockSpec((d, d), lambda i: (0, 0)),
            pl.BlockSpec((d, d), lambda i: (0, 0)),
            pl.BlockSpec((d, d), lambda i: (0, 0)),
            pl.BlockSpec((9, d), lambda i: (0, 0)),
            pl.BlockSpec(memory_space=pltpu.SMEM),
        ],
        out_specs=pl.BlockSpec((rb, d), lambda i: (i, 0)),
        out_shape=jax.ShapeDtypeStruct((n, d), jnp.float32),
    )(node, partials, partials, W1, W2, W3, vecs, eps2)
    return out


# speedup vs baseline: 1.0158x; 1.0158x over previous
import functools

import jax
import jax.numpy as jnp
from jax import lax
from jax.experimental import pallas as pl
from jax.experimental.pallas import tpu as pltpu
from jax.experimental.pallas import tpu_sc as plsc

LANES = 128
NW = 32


NBUF = 2
IBUF = 4

FASTC = 1
GPF = 136
GPS = 24


def _sc_aggregate(node, e2, n_acc, gpf, gps):
    n, d = node.shape
    rpt = n_acc // 16
    nzc = rpt // LANES
    mesh = plsc.VectorSubcoreMesh(core_axis_name="c", subcore_axis_name="s")

    @functools.partial(
        pl.kernel,
        out_type=jax.ShapeDtypeStruct((2, n_acc, d), jnp.float32),
        mesh=mesh,
        scratch_types=[
            pltpu.VMEM((IBUF, 2, LANES), jnp.int32),
            pltpu.VMEM((NBUF, LANES, d), jnp.float32),
            pltpu.VMEM_SHARED((n_acc, d), jnp.float32),
        ] + [pltpu.SemaphoreType.DMA] * (NBUF + IBUF),
    )
    def k(node_hbm, e2_hbm, out_hbm, idx, rows, aggr, *sems):
        gsem = sems[:NBUF]
        isem = sems[NBUF:]
        c = lax.axis_index("c")
        s = lax.axis_index("s")
        fast = c == FASTC
        gcnt = jnp.where(fast, gpf, gps)
        gbase = jnp.where(fast, s * gpf, 16 * gpf + s * gps)

        for q in range(IBUF):
            pltpu.async_copy(e2_hbm.at[gbase + q], idx.at[q], isem[q])

        def zrow(r, carry):
            for j in range(d // 16):
                rows[0, r, pl.ds(j * 16, 16)] = jnp.zeros((16,), jnp.float32)
            return carry
        lax.fori_loop(0, LANES, zrow, 0)
        for kk in range(nzc):
            pltpu.sync_copy(rows.at[0],
                            aggr.at[pl.ds(s * rpt + kk * LANES, LANES)])

        for b in range(NBUF):
            pltpu.make_async_copy(e2_hbm.at[gbase + b], idx.at[b],
                                  isem[b]).wait()
            pltpu.async_copy(node_hbm.at[idx.at[b, 0]], rows.at[b], gsem[b])
        plsc.subcore_barrier()

        def step(g, j, fetch_ahead, gather_ahead):
            b = j % NBUF
            q = j % IBUF
            pltpu.make_async_copy(node_hbm.at[idx.at[q, 0]],
                                  rows.at[b], gsem[b]).wait()
            pltpu.sync_copy(rows.at[b], aggr.at[idx.at[q, 1]], add=True)
            if fetch_ahead:
                pltpu.async_copy(e2_hbm.at[gbase + g + IBUF], idx.at[q],
                                 isem[q])
            if gather_ahead:
                q2 = (j + NBUF) % IBUF
                pltpu.make_async_copy(e2_hbm.at[gbase + g + NBUF],
                                      idx.at[q2], isem[q2]).wait()
                pltpu.async_copy(node_hbm.at[idx.at[q2, 0]], rows.at[b],
                                 gsem[b])

        def body(kk, carry):
            for j in range(IBUF):
                step(kk * IBUF + j, j, True, True)
            return carry
        lax.fori_loop(0, gcnt // IBUF - 1, body, 0)
        for j in range(IBUF):
            g = gcnt - IBUF + j
            step(g, j, False, j + NBUF < IBUF)

        plsc.subcore_barrier()
        pltpu.sync_copy(aggr.at[pl.ds(s * rpt, rpt)],
                        out_hbm.at[c, pl.ds(s * rpt, rpt)])

    return k(node, e2)


def _mlp_block(node_ref, p0_ref, p1_ref, w1_ref, w2_ref, w3_ref, v_ref,
               eps_ref, out_ref):
    def ln(x, g, b):
        mu = jnp.mean(x, axis=-1, keepdims=True)
        var = jnp.mean((x - mu) ** 2, axis=-1, keepdims=True)
        return (x - mu) * lax.rsqrt(var + 1e-5) * g + b

    b1 = v_ref[0:1, :]
    g1 = v_ref[1:2, :]
    be1 = v_ref[2:3, :]
    b2 = v_ref[3:4, :]
    g2 = v_ref[4:5, :]
    be2 = v_ref[5:6, :]
    b3 = v_ref[6:7, :]
    gn = v_ref[7:8, :]
    bn = v_ref[8:9, :]

    h = (1.0 + eps_ref[0]) * node_ref[...] + p0_ref[0] + p1_ref[0]
    x = jnp.dot(h, w1_ref[...], preferred_element_type=jnp.float32) + b1
    x = jnp.maximum(ln(x, g1, be1), 0.0)
    x = jnp.dot(x, w2_ref[...], preferred_element_type=jnp.float32) + b2
    x = jnp.maximum(ln(x, g2, be2), 0.0)
    x = jnp.dot(x, w3_ref[...], preferred_element_type=jnp.float32) + b3
    out_ref[...] = jnp.maximum(ln(x, gn, bn), 0.0)


def kernel(node, edge_index, edge_attr, batch_ptr, W1, b1, g1, be1,
           W2, b2, g2, be2, W3, b3, eps, gN, bN):
    n, d = node.shape
    e = edge_index.shape[1]

    scale = -(-e // (16 * (GPF + GPS) * LANES))
    gpf, gps = GPF * scale, GPS * scale
    gt = 16 * (gpf + gps)
    pad = gt * LANES - e
    src = edge_index[0].astype(jnp.int32)
    dst = edge_index[1].astype(jnp.int32)
    if pad:
        src = jnp.concatenate([src, jnp.zeros((pad,), jnp.int32)])
        dst = jnp.concatenate([dst, jnp.full((pad,), n, jnp.int32)])
    e2 = jnp.stack([src.reshape(gt, LANES), dst.reshape(gt, LANES)], axis=1)

    n_acc = -(-(n + 1) // (16 * LANES)) * (16 * LANES)

    partials = _sc_aggregate(node, e2, n_acc, gpf, gps)

    rb = 1000
    grid = (n // rb,)
    vecs = jnp.stack([b1, g1, be1, b2, g2, be2, b3, gN, bN])
    eps2 = jnp.reshape(eps, (1,))

    out = pl.pallas_call(
        _mlp_block,
        grid=grid,
        in_specs=[
            pl.BlockSpec((rb, d), lambda i: (i, 0)),
            pl.BlockSpec((1, rb, d), lambda i: (0, i, 0)),
            pl.BlockSpec((1, rb, d), lambda i: (1, i, 0)),
            pl.BlockSpec((d, d), lambda i: (0, 0)),
            pl.BlockSpec((d, d), lambda i: (0, 0)),
            pl.BlockSpec((d, d), lambda i: (0, 0)),
            pl.BlockSpec((9, d), lambda i: (0, 0)),
            pl.BlockSpec(memory_space=pltpu.SMEM),
        ],
        out_specs=pl.BlockSpec((rb, d), lambda i: (i, 0)),
        out_shape=jax.ShapeDtypeStruct((n, d), jnp.float32),
    )(node, partials, partials, W1, W2, W3, vecs, eps2)
    return out


# speedup vs baseline: 1.0999x; 1.0828x over previous
import functools

import jax
import jax.numpy as jnp
from jax import lax
from jax.experimental import pallas as pl
from jax.experimental.pallas import tpu as pltpu
from jax.experimental.pallas import tpu_sc as plsc

LANES = 128
NW = 32


NBUF = 2
IBUF = 4

FASTC = 1
GPF = 144
GPS = 16


def _sc_aggregate(node, e2, n_acc, gpf, gps):
    n, d = node.shape
    rpt = n_acc // 16
    nzc = rpt // LANES
    mesh = plsc.VectorSubcoreMesh(core_axis_name="c", subcore_axis_name="s")

    @functools.partial(
        pl.kernel,
        out_type=jax.ShapeDtypeStruct((2, n_acc, d), jnp.float32),
        mesh=mesh,
        scratch_types=[
            pltpu.VMEM((IBUF, 2, LANES), jnp.int32),
            pltpu.VMEM((NBUF, LANES, d), jnp.float32),
            pltpu.VMEM_SHARED((n_acc, d), jnp.float32),
        ] + [pltpu.SemaphoreType.DMA] * (NBUF + IBUF),
    )
    def k(node_hbm, e2_hbm, out_hbm, idx, rows, aggr, *sems):
        gsem = sems[:NBUF]
        isem = sems[NBUF:]
        c = lax.axis_index("c")
        s = lax.axis_index("s")
        fast = c == FASTC
        gcnt = jnp.where(fast, gpf, gps)
        gbase = jnp.where(fast, s * gpf, 16 * gpf + s * gps)

        for q in range(IBUF):
            pltpu.async_copy(e2_hbm.at[gbase + q], idx.at[q], isem[q])

        def zrow(r, carry):
            for j in range(d // 16):
                rows[0, r, pl.ds(j * 16, 16)] = jnp.zeros((16,), jnp.float32)
            return carry
        lax.fori_loop(0, LANES, zrow, 0)
        for kk in range(nzc):
            pltpu.sync_copy(rows.at[0],
                            aggr.at[pl.ds(s * rpt + kk * LANES, LANES)])

        for b in range(NBUF):
            pltpu.make_async_copy(e2_hbm.at[gbase + b], idx.at[b],
                                  isem[b]).wait()
            pltpu.async_copy(node_hbm.at[idx.at[b, 0]], rows.at[b], gsem[b])
        plsc.subcore_barrier()

        def step(g, j, fetch_ahead, gather_ahead):
            b = j % NBUF
            q = j % IBUF
            pltpu.make_async_copy(node_hbm.at[idx.at[q, 0]],
                                  rows.at[b], gsem[b]).wait()
            pltpu.sync_copy(rows.at[b], aggr.at[idx.at[q, 1]], add=True)
            if fetch_ahead:
                pltpu.async_copy(e2_hbm.at[gbase + g + IBUF], idx.at[q],
                                 isem[q])
            if gather_ahead:
                q2 = (j + NBUF) % IBUF
                pltpu.make_async_copy(e2_hbm.at[gbase + g + NBUF],
                                      idx.at[q2], isem[q2]).wait()
                pltpu.async_copy(node_hbm.at[idx.at[q2, 0]], rows.at[b],
                                 gsem[b])

        def body(kk, carry):
            for j in range(IBUF):
                step(kk * IBUF + j, j, True, True)
            return carry
        lax.fori_loop(0, gcnt // IBUF - 1, body, 0)
        for j in range(IBUF):
            g = gcnt - IBUF + j
            step(g, j, False, j + NBUF < IBUF)

        plsc.subcore_barrier()
        pltpu.sync_copy(aggr.at[pl.ds(s * rpt, rpt)],
                        out_hbm.at[c, pl.ds(s * rpt, rpt)])

    return k(node, e2)


def _mlp_block(node_ref, p0_ref, p1_ref, w1_ref, w2_ref, w3_ref, v_ref,
               eps_ref, out_ref):
    def ln(x, g, b):
        mu = jnp.mean(x, axis=-1, keepdims=True)
        var = jnp.mean((x - mu) ** 2, axis=-1, keepdims=True)
        return (x - mu) * lax.rsqrt(var + 1e-5) * g + b

    b1 = v_ref[0:1, :]
    g1 = v_ref[1:2, :]
    be1 = v_ref[2:3, :]
    b2 = v_ref[3:4, :]
    g2 = v_ref[4:5, :]
    be2 = v_ref[5:6, :]
    b3 = v_ref[6:7, :]
    gn = v_ref[7:8, :]
    bn = v_ref[8:9, :]

    h = (1.0 + eps_ref[0]) * node_ref[...] + p0_ref[0] + p1_ref[0]
    x = jnp.dot(h, w1_ref[...], preferred_element_type=jnp.float32) + b1
    x = jnp.maximum(ln(x, g1, be1), 0.0)
    x = jnp.dot(x, w2_ref[...], preferred_element_type=jnp.float32) + b2
    x = jnp.maximum(ln(x, g2, be2), 0.0)
    x = jnp.dot(x, w3_ref[...], preferred_element_type=jnp.float32) + b3
    out_ref[...] = jnp.maximum(ln(x, gn, bn), 0.0)


def kernel(node, edge_index, edge_attr, batch_ptr, W1, b1, g1, be1,
           W2, b2, g2, be2, W3, b3, eps, gN, bN):
    n, d = node.shape
    e = edge_index.shape[1]

    scale = -(-e // (16 * (GPF + GPS) * LANES))
    gpf, gps = GPF * scale, GPS * scale
    gt = 16 * (gpf + gps)
    pad = gt * LANES - e
    src = edge_index[0].astype(jnp.int32)
    dst = edge_index[1].astype(jnp.int32)
    if pad:
        src = jnp.concatenate([src, jnp.zeros((pad,), jnp.int32)])
        dst = jnp.concatenate([dst, jnp.full((pad,), n, jnp.int32)])
    e2 = jnp.stack([src.reshape(gt, LANES), dst.reshape(gt, LANES)], axis=1)

    n_acc = -(-(n + 1) // (16 * LANES)) * (16 * LANES)

    partials = _sc_aggregate(node, e2, n_acc, gpf, gps)

    rb = 1000
    grid = (n // rb,)
    vecs = jnp.stack([b1, g1, be1, b2, g2, be2, b3, gN, bN])
    eps2 = jnp.reshape(eps, (1,))

    out = pl.pallas_call(
        _mlp_block,
        grid=grid,
        in_specs=[
            pl.BlockSpec((rb, d), lambda i: (i, 0)),
            pl.BlockSpec((1, rb, d), lambda i: (0, i, 0)),
            pl.BlockSpec((1, rb, d), lambda i: (1, i, 0)),
            pl.BlockSpec((d, d), lambda i: (0, 0)),
            pl.BlockSpec((d, d), lambda i: (0, 0)),
            pl.BlockSpec((d, d), lambda i: (0, 0)),
            pl.BlockSpec((9, d), lambda i: (0, 0)),
            pl.BlockSpec(memory_space=pltpu.SMEM),
        ],
        out_specs=pl.BlockSpec((rb, d), lambda i: (i, 0)),
        out_shape=jax.ShapeDtypeStruct((n, d), jnp.float32),
    )(node, partials, partials, W1, W2, W3, vecs, eps2)
    return out


# speedup vs baseline: 1.1085x; 1.0079x over previous
import functools

import jax
import jax.numpy as jnp
from jax import lax
from jax.experimental import pallas as pl
from jax.experimental.pallas import tpu as pltpu
from jax.experimental.pallas import tpu_sc as plsc

LANES = 128
NW = 32


NBUF = 2
IBUF = 4

FASTC = 1
GPF = 152
GPS = 8


def _sc_aggregate(node, e2, n_acc, gpf, gps):
    n, d = node.shape
    rpt = n_acc // 16
    nzc = rpt // LANES
    mesh = plsc.VectorSubcoreMesh(core_axis_name="c", subcore_axis_name="s")

    @functools.partial(
        pl.kernel,
        out_type=jax.ShapeDtypeStruct((2, n_acc, d), jnp.float32),
        mesh=mesh,
        scratch_types=[
            pltpu.VMEM((IBUF, 2, LANES), jnp.int32),
            pltpu.VMEM((NBUF, LANES, d), jnp.float32),
            pltpu.VMEM_SHARED((n_acc, d), jnp.float32),
        ] + [pltpu.SemaphoreType.DMA] * (NBUF + IBUF),
    )
    def k(node_hbm, e2_hbm, out_hbm, idx, rows, aggr, *sems):
        gsem = sems[:NBUF]
        isem = sems[NBUF:]
        c = lax.axis_index("c")
        s = lax.axis_index("s")
        fast = c == FASTC
        gcnt = jnp.where(fast, gpf, gps)
        gbase = jnp.where(fast, s * gpf, 16 * gpf + s * gps)

        for q in range(IBUF):
            pltpu.async_copy(e2_hbm.at[gbase + q], idx.at[q], isem[q])

        def zrow(r, carry):
            for j in range(d // 16):
                rows[0, r, pl.ds(j * 16, 16)] = jnp.zeros((16,), jnp.float32)
            return carry
        lax.fori_loop(0, LANES, zrow, 0)
        for kk in range(nzc):
            pltpu.sync_copy(rows.at[0],
                            aggr.at[pl.ds(s * rpt + kk * LANES, LANES)])

        for b in range(NBUF):
            pltpu.make_async_copy(e2_hbm.at[gbase + b], idx.at[b],
                                  isem[b]).wait()
            pltpu.async_copy(node_hbm.at[idx.at[b, 0]], rows.at[b], gsem[b])
        plsc.subcore_barrier()

        def step(g, j, fetch_ahead, gather_ahead):
            b = j % NBUF
            q = j % IBUF
            pltpu.make_async_copy(node_hbm.at[idx.at[q, 0]],
                                  rows.at[b], gsem[b]).wait()
            pltpu.sync_copy(rows.at[b], aggr.at[idx.at[q, 1]], add=True)
            if fetch_ahead:
                pltpu.async_copy(e2_hbm.at[gbase + g + IBUF], idx.at[q],
                                 isem[q])
            if gather_ahead:
                q2 = (j + NBUF) % IBUF
                pltpu.make_async_copy(e2_hbm.at[gbase + g + NBUF],
                                      idx.at[q2], isem[q2]).wait()
                pltpu.async_copy(node_hbm.at[idx.at[q2, 0]], rows.at[b],
                                 gsem[b])

        def body(kk, carry):
            for j in range(IBUF):
                step(kk * IBUF + j, j, True, True)
            return carry
        lax.fori_loop(0, gcnt // IBUF - 1, body, 0)
        for j in range(IBUF):
            g = gcnt - IBUF + j
            step(g, j, False, j + NBUF < IBUF)

        plsc.subcore_barrier()
        pltpu.sync_copy(aggr.at[pl.ds(s * rpt, rpt)],
                        out_hbm.at[c, pl.ds(s * rpt, rpt)])

    return k(node, e2)


def _mlp_block(node_ref, p0_ref, p1_ref, w1_ref, w2_ref, w3_ref, v_ref,
               eps_ref, out_ref):
    def ln(x, g, b):
        mu = jnp.mean(x, axis=-1, keepdims=True)
        var = jnp.mean((x - mu) ** 2, axis=-1, keepdims=True)
        return (x - mu) * lax.rsqrt(var + 1e-5) * g + b

    b1 = v_ref[0:1, :]
    g1 = v_ref[1:2, :]
    be1 = v_ref[2:3, :]
    b2 = v_ref[3:4, :]
    g2 = v_ref[4:5, :]
    be2 = v_ref[5:6, :]
    b3 = v_ref[6:7, :]
    gn = v_ref[7:8, :]
    bn = v_ref[8:9, :]

    h = (1.0 + eps_ref[0]) * node_ref[...] + p0_ref[0] + p1_ref[0]
    x = jnp.dot(h, w1_ref[...], preferred_element_type=jnp.float32) + b1
    x = jnp.maximum(ln(x, g1, be1), 0.0)
    x = jnp.dot(x, w2_ref[...], preferred_element_type=jnp.float32) + b2
    x = jnp.maximum(ln(x, g2, be2), 0.0)
    x = jnp.dot(x, w3_ref[...], preferred_element_type=jnp.float32) + b3
    out_ref[...] = jnp.maximum(ln(x, gn, bn), 0.0)


def kernel(node, edge_index, edge_attr, batch_ptr, W1, b1, g1, be1,
           W2, b2, g2, be2, W3, b3, eps, gN, bN):
    n, d = node.shape
    e = edge_index.shape[1]

    scale = -(-e // (16 * (GPF + GPS) * LANES))
    gpf, gps = GPF * scale, GPS * scale
    gt = 16 * (gpf + gps)
    pad = gt * LANES - e
    src = edge_index[0].astype(jnp.int32)
    dst = edge_index[1].astype(jnp.int32)
    if pad:
        src = jnp.concatenate([src, jnp.zeros((pad,), jnp.int32)])
        dst = jnp.concatenate([dst, jnp.full((pad,), n, jnp.int32)])
    e2 = jnp.stack([src.reshape(gt, LANES), dst.reshape(gt, LANES)], axis=1)

    n_acc = -(-(n + 1) // (16 * LANES)) * (16 * LANES)

    partials = _sc_aggregate(node, e2, n_acc, gpf, gps)

    rb = 1000
    grid = (n // rb,)
    vecs = jnp.stack([b1, g1, be1, b2, g2, be2, b3, gN, bN])
    eps2 = jnp.reshape(eps, (1,))

    out = pl.pallas_call(
        _mlp_block,
        grid=grid,
        in_specs=[
            pl.BlockSpec((rb, d), lambda i: (i, 0)),
            pl.BlockSpec((1, rb, d), lambda i: (0, i, 0)),
            pl.BlockSpec((1, rb, d), lambda i: (1, i, 0)),
            pl.BlockSpec((d, d), lambda i: (0, 0)),
            pl.BlockSpec((d, d), lambda i: (0, 0)),
            pl.BlockSpec((d, d), lambda i: (0, 0)),
            pl.BlockSpec((9, d), lambda i: (0, 0)),
            pl.BlockSpec(memory_space=pltpu.SMEM),
        ],
        out_specs=pl.BlockSpec((rb, d), lambda i: (i, 0)),
        out_shape=jax.ShapeDtypeStruct((n, d), jnp.float32),
    )(node, partials, partials, W1, W2, W3, vecs, eps2)
    return out


# speedup vs baseline: 1.1108x; 1.0020x over previous
import functools

import jax
import jax.numpy as jnp
from jax import lax
from jax.experimental import pallas as pl
from jax.experimental.pallas import tpu as pltpu
from jax.experimental.pallas import tpu_sc as plsc

LANES = 128
NW = 32


NBUF = 2
IBUF = 4

FASTC = 0
GPF = 152
GPS = 8


def _sc_aggregate(node, e2, n_acc, gpf, gps):
    n, d = node.shape
    rpt = n_acc // 16
    nzc = rpt // LANES
    mesh = plsc.VectorSubcoreMesh(core_axis_name="c", subcore_axis_name="s")

    @functools.partial(
        pl.kernel,
        out_type=jax.ShapeDtypeStruct((2, n_acc, d), jnp.float32),
        mesh=mesh,
        scratch_types=[
            pltpu.VMEM((IBUF, 2, LANES), jnp.int32),
            pltpu.VMEM((NBUF, LANES, d), jnp.float32),
            pltpu.VMEM_SHARED((n_acc, d), jnp.float32),
        ] + [pltpu.SemaphoreType.DMA] * (NBUF + IBUF),
    )
    def k(node_hbm, e2_hbm, out_hbm, idx, rows, aggr, *sems):
        gsem = sems[:NBUF]
        isem = sems[NBUF:]
        c = lax.axis_index("c")
        s = lax.axis_index("s")
        fast = c == FASTC
        gcnt = jnp.where(fast, gpf, gps)
        gbase = jnp.where(fast, s * gpf, 16 * gpf + s * gps)

        for q in range(IBUF):
            pltpu.async_copy(e2_hbm.at[gbase + q], idx.at[q], isem[q])

        def zrow(r, carry):
            for j in range(d // 16):
                rows[0, r, pl.ds(j * 16, 16)] = jnp.zeros((16,), jnp.float32)
            return carry
        lax.fori_loop(0, LANES, zrow, 0)
        for kk in range(nzc):
            pltpu.sync_copy(rows.at[0],
                            aggr.at[pl.ds(s * rpt + kk * LANES, LANES)])

        for b in range(NBUF):
            pltpu.make_async_copy(e2_hbm.at[gbase + b], idx.at[b],
                                  isem[b]).wait()
            pltpu.async_copy(node_hbm.at[idx.at[b, 0]], rows.at[b], gsem[b])
        plsc.subcore_barrier()

        def step(g, j, fetch_ahead, gather_ahead):
            b = j % NBUF
            q = j % IBUF
            pltpu.make_async_copy(node_hbm.at[idx.at[q, 0]],
                                  rows.at[b], gsem[b]).wait()
            pltpu.sync_copy(rows.at[b], aggr.at[idx.at[q, 1]], add=True)
            if fetch_ahead:
                pltpu.async_copy(e2_hbm.at[gbase + g + IBUF], idx.at[q],
                                 isem[q])
            if gather_ahead:
                q2 = (j + NBUF) % IBUF
                pltpu.make_async_copy(e2_hbm.at[gbase + g + NBUF],
                                      idx.at[q2], isem[q2]).wait()
                pltpu.async_copy(node_hbm.at[idx.at[q2, 0]], rows.at[b],
                                 gsem[b])

        def body(kk, carry):
            for j in range(IBUF):
                step(kk * IBUF + j, j, True, True)
            return carry
        lax.fori_loop(0, gcnt // IBUF - 1, body, 0)
        for j in range(IBUF):
            g = gcnt - IBUF + j
            step(g, j, False, j + NBUF < IBUF)

        plsc.subcore_barrier()
        pltpu.sync_copy(aggr.at[pl.ds(s * rpt, rpt)],
                        out_hbm.at[c, pl.ds(s * rpt, rpt)])

    return k(node, e2)


def _mlp_block(node_ref, p0_ref, p1_ref, w1_ref, w2_ref, w3_ref, v_ref,
               eps_ref, out_ref):
    def ln(x, g, b):
        mu = jnp.mean(x, axis=-1, keepdims=True)
        var = jnp.mean((x - mu) ** 2, axis=-1, keepdims=True)
        return (x - mu) * lax.rsqrt(var + 1e-5) * g + b

    b1 = v_ref[0:1, :]
    g1 = v_ref[1:2, :]
    be1 = v_ref[2:3, :]
    b2 = v_ref[3:4, :]
    g2 = v_ref[4:5, :]
    be2 = v_ref[5:6, :]
    b3 = v_ref[6:7, :]
    gn = v_ref[7:8, :]
    bn = v_ref[8:9, :]

    h = (1.0 + eps_ref[0]) * node_ref[...] + p0_ref[0] + p1_ref[0]
    x = jnp.dot(h, w1_ref[...], preferred_element_type=jnp.float32) + b1
    x = jnp.maximum(ln(x, g1, be1), 0.0)
    x = jnp.dot(x, w2_ref[...], preferred_element_type=jnp.float32) + b2
    x = jnp.maximum(ln(x, g2, be2), 0.0)
    x = jnp.dot(x, w3_ref[...], preferred_element_type=jnp.float32) + b3
    out_ref[...] = jnp.maximum(ln(x, gn, bn), 0.0)


def kernel(node, edge_index, edge_attr, batch_ptr, W1, b1, g1, be1,
           W2, b2, g2, be2, W3, b3, eps, gN, bN):
    n, d = node.shape
    e = edge_index.shape[1]

    scale = -(-e // (16 * (GPF + GPS) * LANES))
    gpf, gps = GPF * scale, GPS * scale
    gt = 16 * (gpf + gps)
    pad = gt * LANES - e
    src = edge_index[0].astype(jnp.int32)
    dst = edge_index[1].astype(jnp.int32)
    if pad:
        src = jnp.concatenate([src, jnp.zeros((pad,), jnp.int32)])
        dst = jnp.concatenate([dst, jnp.full((pad,), n, jnp.int32)])
    e2 = jnp.stack([src.reshape(gt, LANES), dst.reshape(gt, LANES)], axis=1)

    n_acc = -(-(n + 1) // (16 * LANES)) * (16 * LANES)

    partials = _sc_aggregate(node, e2, n_acc, gpf, gps)

    rb = 1000
    grid = (n // rb,)
    vecs = jnp.stack([b1, g1, be1, b2, g2, be2, b3, gN, bN])
    eps2 = jnp.reshape(eps, (1,))

    out = pl.pallas_call(
        _mlp_block,
        grid=grid,
        in_specs=[
            pl.BlockSpec((rb, d), lambda i: (i, 0)),
            pl.BlockSpec((1, rb, d), lambda i: (0, i, 0)),
            pl.BlockSpec((1, rb, d), lambda i: (1, i, 0)),
            pl.BlockSpec((d, d), lambda i: (0, 0)),
            pl.BlockSpec((d, d), lambda i: (0, 0)),
            pl.BlockSpec((d, d), lambda i: (0, 0)),
            pl.BlockSpec((9, d), lambda i: (0, 0)),
            pl.BlockSpec(memory_space=pltpu.SMEM),
        ],
        out_specs=pl.BlockSpec((rb, d), lambda i: (i, 0)),
        out_shape=jax.ShapeDtypeStruct((n, d), jnp.float32),
    )(node, partials, partials, W1, W2, W3, vecs, eps2)
    return out
